# baseline (speedup 1.0000x reference)
import functools

import jax
import jax.numpy as jnp
from jax import lax
from jax.experimental import pallas as pl
from jax.experimental.pallas import tpu as pltpu
from jax.experimental.pallas import tpu_sc as plsc

NC = 2
NS = 16

N_NODES = 10000
F_DIM = 128
FH = F_DIM // NC
CHUNK = 80
SEC = 12
ACC_ROWS = 10112
STRIPE = ACC_ROWS // NS
H2_STRIPE = N_NODES // NS
TRASH = N_NODES
HIST_W = 8

_mesh = plsc.VectorSubcoreMesh(core_axis_name="c", subcore_axis_name="s")
_untiled = pltpu.CompilerParams(use_tc_tiling_on_sc=False)


def _sc_hist(ei2, ones_h, zeros_h, kch):
    hk = kch // 2

    @functools.partial(
        pl.kernel,
        out_type=jax.ShapeDtypeStruct((ACC_ROWS, F_DIM), jnp.float32),
        mesh=_mesh,
        compiler_params=_untiled,
        scratch_types=[
            pltpu.VMEM((hk, CHUNK), jnp.int32),
            pltpu.VMEM((CHUNK, HIST_W), jnp.float32),
            pltpu.VMEM_SHARED((ACC_ROWS, HIST_W), jnp.float32),
            pltpu.SemaphoreType.DMA,
        ],
    )
    def hist(ei_h, one_h, zer_h, degp_h, dst_v, one_v, deg_s, hsem):
        c = lax.axis_index("c")
        s = lax.axis_index("s")
        pltpu.sync_copy(ei_h.at[1, s, pl.ds(c * hk, hk)], dst_v)
        pltpu.sync_copy(one_h, one_v)
        pltpu.sync_copy(zer_h, deg_s.at[pl.ds(s * STRIPE, STRIPE)])
        plsc.subcore_barrier()

        pltpu.async_copy(one_v, deg_s.at[dst_v.at[0]], hsem, add=True)

        def body(j, carry):
            pltpu.async_copy(one_v, deg_s.at[dst_v.at[j]], hsem, add=True)
            pltpu.make_async_copy(one_v, deg_s.at[dst_v.at[0]], hsem).wait()
            return carry

        lax.fori_loop(1, hk, body, 0)
        pltpu.make_async_copy(one_v, deg_s.at[dst_v.at[0]], hsem).wait()
        plsc.subcore_barrier()
        pltpu.sync_copy(
            deg_s.at[pl.ds(s * STRIPE, STRIPE)],
            degp_h.at[pl.ds(s * STRIPE, STRIPE), pl.ds(c * HIST_W, HIST_W)],
        )

    return hist(ei2, ones_h, zeros_h)


def _sc_scatter(ei2, h2, zeros_h, kch):
    nsec = kch // SEC
    quads = SEC // 4

    @functools.partial(
        pl.kernel,
        out_type=jax.ShapeDtypeStruct((ACC_ROWS, F_DIM), jnp.float32),
        mesh=_mesh,
        compiler_params=_untiled,
        scratch_types=[
            pltpu.VMEM((kch, CHUNK), jnp.int32),
            pltpu.VMEM((2, SEC, CHUNK), jnp.int32),
            pltpu.VMEM((4, CHUNK, FH), jnp.float32),
            pltpu.VMEM_SHARED((N_NODES, FH), jnp.float32),
            pltpu.VMEM_SHARED((ACC_ROWS, FH), jnp.float32),
            [pltpu.SemaphoreType.DMA] * 4,
            [pltpu.SemaphoreType.DMA] * 4,
            pltpu.SemaphoreType.DMA,
        ],
    )
    def scat(ei_h, h2_h, zer_h, accp_h,
             src_v, dst_sec, bufs, h2_s, acc_s, gsems, ssems, dsem):
        c = lax.axis_index("c")
        s = lax.axis_index("s")
        pltpu.sync_copy(
            h2_h.at[pl.ds(s * H2_STRIPE, H2_STRIPE), pl.ds(c * FH, FH)],
            h2_s.at[pl.ds(s * H2_STRIPE, H2_STRIPE)])
        pltpu.sync_copy(ei_h.at[0, s], src_v)
        pltpu.async_copy(ei_h.at[1, s, pl.ds(0, SEC)], dst_sec.at[0], dsem)
        pltpu.sync_copy(zer_h, acc_s.at[pl.ds(s * STRIPE, STRIPE)])
        plsc.subcore_barrier()
        pltpu.make_async_copy(
            ei_h.at[1, s, pl.ds(0, SEC)], dst_sec.at[0], dsem).wait()

        for k in range(4):
            pltpu.async_copy(h2_s.at[src_v.at[k]], bufs.at[k], gsems[k])

        def outer(sec, carry):
            p = sec % 2
            sn = jnp.where(sec + 1 < nsec, sec + 1, 0)
            pltpu.async_copy(
                ei_h.at[1, s, pl.ds(sn * SEC, SEC)], dst_sec.at[1 - p], dsem)

            def inner(q, icarry):
                j = sec * SEC + q * 4
                for k in range(4):
                    pltpu.make_async_copy(
                        h2_s.at[src_v.at[0]], bufs.at[k], gsems[k]).wait()
                    pltpu.async_copy(
                        bufs.at[k], acc_s.at[dst_sec.at[p, q * 4 + k]],
                        ssems[k], add=True)
                for k in range(4):
                    jn = jnp.where(j + 4 + k < kch, j + 4 + k, 0)
                    pltpu.make_async_copy(
                        bufs.at[k], acc_s.at[dst_sec.at[p, 0]],
                        ssems[k]).wait()
                    pltpu.async_copy(
                        h2_s.at[src_v.at[jn]], bufs.at[k], gsems[k])
                return icarry

            lax.fori_loop(0, quads, inner, carry)
            pltpu.make_async_copy(
                ei_h.at[1, s, pl.ds(sn * SEC, SEC)], dst_sec.at[1 - p],
                dsem).wait()
            return carry

        lax.fori_loop(0, nsec, outer, 0)
        for k in range(4):
            pltpu.make_async_copy(
                h2_s.at[src_v.at[0]], bufs.at[k], gsems[k]).wait()
        plsc.subcore_barrier()
        pltpu.sync_copy(
            acc_s.at[pl.ds(s * STRIPE, STRIPE)],
            accp_h.at[pl.ds(s * STRIPE, STRIPE), pl.ds(c * FH, FH)],
        )

    return scat(ei2, h2, zeros_h)


def _tc_matmul(x, W):
    blk = 1000
    grid = N_NODES // blk

    def body(x_ref, w_ref, o_ref):
        o_ref[...] = jnp.dot(
            x_ref[...], w_ref[...], preferred_element_type=jnp.float32)

    return pl.pallas_call(
        body,
        grid=(grid,),
        in_specs=[
            pl.BlockSpec((blk, F_DIM), lambda i: (i, 0)),
            pl.BlockSpec((F_DIM, F_DIM), lambda i: (0, 0)),
        ],
        out_specs=pl.BlockSpec((blk, F_DIM), lambda i: (i, 0)),
        out_shape=jax.ShapeDtypeStruct((N_NODES, F_DIM), jnp.float32),
    )(x, W)


def _tc_scale(h, degp):
    blk = 1000
    grid = N_NODES // blk

    def body(h_ref, d_ref, o_ref):
        deg = d_ref[:, 0:1] + d_ref[:, HIST_W:HIST_W + 1] + 1.0
        o_ref[...] = h_ref[...] * lax.rsqrt(deg)

    return pl.pallas_call(
        body,
        grid=(grid,),
        in_specs=[
            pl.BlockSpec((blk, F_DIM), lambda i: (i, 0)),
            pl.BlockSpec((blk, F_DIM), lambda i: (i, 0)),
        ],
        out_specs=pl.BlockSpec((blk, F_DIM), lambda i: (i, 0)),
        out_shape=jax.ShapeDtypeStruct((N_NODES, F_DIM), jnp.float32),
    )(h, degp)


def _tc_final(acc, h2, degp, b2):
    blk = 1000
    grid = N_NODES // blk

    def body(a_ref, h2_ref, d_ref, b_ref, o_ref):
        deg = d_ref[:, 0:1] + d_ref[:, HIST_W:HIST_W + 1] + 1.0
        dinv = lax.rsqrt(deg)
        o_ref[...] = dinv * (a_ref[...] + h2_ref[...]) + b_ref[...]

    return pl.pallas_call(
        body,
        grid=(grid,),
        in_specs=[
            pl.BlockSpec((blk, F_DIM), lambda i: (i, 0)),
            pl.BlockSpec((blk, F_DIM), lambda i: (i, 0)),
            pl.BlockSpec((blk, F_DIM), lambda i: (i, 0)),
            pl.BlockSpec((1, F_DIM), lambda i: (0, 0)),
        ],
        out_specs=pl.BlockSpec((blk, F_DIM), lambda i: (i, 0)),
        out_shape=jax.ShapeDtypeStruct((N_NODES, F_DIM), jnp.float32),
    )(acc, h2, degp, b2)


def kernel(x, edge_index, W, b):
    ei = edge_index.astype(jnp.int32)
    e = ei.shape[1]

    epw = -(-e // NS)
    kch = -(-epw // CHUNK)
    kch = -(-kch // SEC) * SEC
    pad = NS * kch * CHUNK - e
    if pad:
        fill = jnp.stack([jnp.zeros((pad,), jnp.int32),
                          jnp.full((pad,), TRASH, jnp.int32)])
        ei = jnp.concatenate([ei, fill], axis=1)
    ei2 = ei.reshape(2, NS, kch, CHUNK)

    ones_h = jnp.ones((CHUNK, HIST_W), jnp.float32)
    zeros_hist = jnp.zeros((STRIPE, HIST_W), jnp.float32)
    zeros_acc = jnp.zeros((STRIPE, FH), jnp.float32)

    h = _tc_matmul(x, W)
    degp = _sc_hist(ei2, ones_h, zeros_hist, kch)
    h2 = _tc_scale(h, degp)
    acc = _sc_scatter(ei2, h2, zeros_acc, kch)
    return _tc_final(acc, h2, degp, b.reshape(1, F_DIM))


# speedup vs baseline: 1.1337x; 1.1337x over previous
import functools

import jax
import jax.numpy as jnp
from jax import lax
from jax.experimental import pallas as pl
from jax.experimental.pallas import tpu as pltpu
from jax.experimental.pallas import tpu_sc as plsc

NC = 2
NS = 16

N_NODES = 10000
F_DIM = 128
FH = F_DIM // NC
CHUNK = 80
SEC = 10
ACC_ROWS = 10112
STRIPE = ACC_ROWS // NS
H2_STRIPE = N_NODES // NS
TRASH = N_NODES
HIST_W = 8

_mesh = plsc.VectorSubcoreMesh(core_axis_name="c", subcore_axis_name="s")
_untiled = pltpu.CompilerParams(use_tc_tiling_on_sc=False)


def _sc_hist(ei2, ones_h, zeros_h, kch):
    hk = kch // 2

    @functools.partial(
        pl.kernel,
        out_type=jax.ShapeDtypeStruct((ACC_ROWS, F_DIM), jnp.float32),
        mesh=_mesh,
        compiler_params=_untiled,
        scratch_types=[
            pltpu.VMEM((hk, CHUNK), jnp.int32),
            pltpu.VMEM((CHUNK, HIST_W), jnp.float32),
            pltpu.VMEM_SHARED((ACC_ROWS, HIST_W), jnp.float32),
            pltpu.SemaphoreType.DMA,
        ],
    )
    def hist(ei_h, one_h, zer_h, degp_h, dst_v, one_v, deg_s, hsem):
        c = lax.axis_index("c")
        s = lax.axis_index("s")
        pltpu.sync_copy(ei_h.at[1, s, pl.ds(c * hk, hk)], dst_v)
        pltpu.sync_copy(one_h, one_v)
        pltpu.sync_copy(zer_h, deg_s.at[pl.ds(s * STRIPE, STRIPE)])
        plsc.subcore_barrier()

        pltpu.async_copy(one_v, deg_s.at[dst_v.at[0]], hsem, add=True)

        def body(j, carry):
            pltpu.async_copy(one_v, deg_s.at[dst_v.at[j]], hsem, add=True)
            pltpu.make_async_copy(one_v, deg_s.at[dst_v.at[0]], hsem).wait()
            return carry

        lax.fori_loop(1, hk, body, 0)
        pltpu.make_async_copy(one_v, deg_s.at[dst_v.at[0]], hsem).wait()
        plsc.subcore_barrier()
        pltpu.sync_copy(
            deg_s.at[pl.ds(s * STRIPE, STRIPE)],
            degp_h.at[pl.ds(s * STRIPE, STRIPE), pl.ds(c * HIST_W, HIST_W)],
        )

    return hist(ei2, ones_h, zeros_h)


def _sc_scatter(ei2, h2, zeros_h, kch):
    nsec = kch // SEC
    pairs = SEC // 2

    @functools.partial(
        pl.kernel,
        out_type=jax.ShapeDtypeStruct((ACC_ROWS, F_DIM), jnp.float32),
        mesh=_mesh,
        compiler_params=_untiled,
        scratch_types=[
            pltpu.VMEM((kch, CHUNK), jnp.int32),
            pltpu.VMEM((2, SEC, CHUNK), jnp.int32),
            pltpu.VMEM((CHUNK, FH), jnp.float32),
            pltpu.VMEM((CHUNK, FH), jnp.float32),
            pltpu.VMEM_SHARED((N_NODES, FH), jnp.float32),
            pltpu.VMEM_SHARED((ACC_ROWS, FH), jnp.float32),
            pltpu.SemaphoreType.DMA,
            pltpu.SemaphoreType.DMA,
            pltpu.SemaphoreType.DMA,
        ],
    )
    def scat(ei_h, h2_h, zer_h, accp_h,
             src_v, dst_sec, buf0, buf1, h2_s, acc_s, sem0, sem1, dsem):
        c = lax.axis_index("c")
        s = lax.axis_index("s")
        pltpu.sync_copy(
            h2_h.at[pl.ds(s * H2_STRIPE, H2_STRIPE), pl.ds(c * FH, FH)],
            h2_s.at[pl.ds(s * H2_STRIPE, H2_STRIPE)])
        pltpu.sync_copy(ei_h.at[0, s], src_v)
        pltpu.async_copy(ei_h.at[1, s, pl.ds(0, SEC)], dst_sec.at[0], dsem)
        pltpu.sync_copy(zer_h, acc_s.at[pl.ds(s * STRIPE, STRIPE)])
        plsc.subcore_barrier()
        pltpu.make_async_copy(
            ei_h.at[1, s, pl.ds(0, SEC)], dst_sec.at[0], dsem).wait()

        pltpu.async_copy(h2_s.at[src_v.at[0]], buf0, sem0)

        def outer(sec, carry):
            p = sec % 2
            sn = jnp.where(sec + 1 < nsec, sec + 1, 0)
            pltpu.async_copy(
                ei_h.at[1, s, pl.ds(sn * SEC, SEC)], dst_sec.at[1 - p], dsem)

            def inner(i, icarry):
                j0 = sec * SEC + 2 * i
                j1 = j0 + 1
                pltpu.async_copy(h2_s.at[src_v.at[j1]], buf1, sem1)
                pltpu.make_async_copy(h2_s.at[src_v.at[0]], buf0, sem0).wait()
                pltpu.sync_copy(buf0, acc_s.at[dst_sec.at[p, 2 * i]], add=True)
                jn = jnp.where(j0 + 2 < kch, j0 + 2, 0)
                pltpu.async_copy(h2_s.at[src_v.at[jn]], buf0, sem0)
                pltpu.make_async_copy(h2_s.at[src_v.at[0]], buf1, sem1).wait()
                pltpu.sync_copy(
                    buf1, acc_s.at[dst_sec.at[p, 2 * i + 1]], add=True)
                return icarry

            lax.fori_loop(0, pairs, inner, carry)
            pltpu.make_async_copy(
                ei_h.at[1, s, pl.ds(sn * SEC, SEC)], dst_sec.at[1 - p],
                dsem).wait()
            return carry

        lax.fori_loop(0, nsec, outer, 0)
        pltpu.make_async_copy(h2_s.at[src_v.at[0]], buf0, sem0).wait()
        plsc.subcore_barrier()
        pltpu.sync_copy(
            acc_s.at[pl.ds(s * STRIPE, STRIPE)],
            accp_h.at[pl.ds(s * STRIPE, STRIPE), pl.ds(c * FH, FH)],
        )

    return scat(ei2, h2, zeros_h)


def _tc_matmul(x, W):
    blk = 1000
    grid = N_NODES // blk

    def body(x_ref, w_ref, o_ref):
        o_ref[...] = jnp.dot(
            x_ref[...], w_ref[...], preferred_element_type=jnp.float32)

    return pl.pallas_call(
        body,
        grid=(grid,),
        in_specs=[
            pl.BlockSpec((blk, F_DIM), lambda i: (i, 0)),
            pl.BlockSpec((F_DIM, F_DIM), lambda i: (0, 0)),
        ],
        out_specs=pl.BlockSpec((blk, F_DIM), lambda i: (i, 0)),
        out_shape=jax.ShapeDtypeStruct((N_NODES, F_DIM), jnp.float32),
    )(x, W)


def _tc_scale(h, degp):
    blk = 1000
    grid = N_NODES // blk

    def body(h_ref, d_ref, o_ref):
        deg = d_ref[:, 0:1] + d_ref[:, HIST_W:HIST_W + 1] + 1.0
        o_ref[...] = h_ref[...] * lax.rsqrt(deg)

    return pl.pallas_call(
        body,
        grid=(grid,),
        in_specs=[
            pl.BlockSpec((blk, F_DIM), lambda i: (i, 0)),
            pl.BlockSpec((blk, F_DIM), lambda i: (i, 0)),
        ],
        out_specs=pl.BlockSpec((blk, F_DIM), lambda i: (i, 0)),
        out_shape=jax.ShapeDtypeStruct((N_NODES, F_DIM), jnp.float32),
    )(h, degp)


def _tc_final(acc, h2, degp, b2):
    blk = 1000
    grid = N_NODES // blk

    def body(a_ref, h2_ref, d_ref, b_ref, o_ref):
        deg = d_ref[:, 0:1] + d_ref[:, HIST_W:HIST_W + 1] + 1.0
        dinv = lax.rsqrt(deg)
        o_ref[...] = dinv * (a_ref[...] + h2_ref[...]) + b_ref[...]

    return pl.pallas_call(
        body,
        grid=(grid,),
        in_specs=[
            pl.BlockSpec((blk, F_DIM), lambda i: (i, 0)),
            pl.BlockSpec((blk, F_DIM), lambda i: (i, 0)),
            pl.BlockSpec((blk, F_DIM), lambda i: (i, 0)),
            pl.BlockSpec((1, F_DIM), lambda i: (0, 0)),
        ],
        out_specs=pl.BlockSpec((blk, F_DIM), lambda i: (i, 0)),
        out_shape=jax.ShapeDtypeStruct((N_NODES, F_DIM), jnp.float32),
    )(acc, h2, degp, b2)


def kernel(x, edge_index, W, b):
    ei = edge_index.astype(jnp.int32)
    e = ei.shape[1]

    epw = -(-e // NS)
    kch = -(-epw // CHUNK)
    kch = -(-kch // SEC) * SEC
    pad = NS * kch * CHUNK - e
    if pad:
        fill = jnp.stack([jnp.zeros((pad,), jnp.int32),
                          jnp.full((pad,), TRASH, jnp.int32)])
        ei = jnp.concatenate([ei, fill], axis=1)
    ei2 = ei.reshape(2, NS, kch, CHUNK)

    ones_h = jnp.ones((CHUNK, HIST_W), jnp.float32)
    zeros_hist = jnp.zeros((STRIPE, HIST_W), jnp.float32)
    zeros_acc = jnp.zeros((STRIPE, FH), jnp.float32)

    h = _tc_matmul(x, W)
    degp = _sc_hist(ei2, ones_h, zeros_hist, kch)
    h2 = _tc_scale(h, degp)
    acc = _sc_scatter(ei2, h2, zeros_acc, kch)
    return _tc_final(acc, h2, degp, b.reshape(1, F_DIM))
